# baseline (speedup 1.0000x reference)
import functools
import math

import jax
import jax.numpy as jnp
from jax import lax
from jax.experimental import pallas as pl
from jax.experimental.pallas import tpu as pltpu
from jax.experimental.pallas import tpu_sc as plsc

N = 10000
E = 320000
D = 128
NP = 10112
ACC_W = 144
QE_W = 80
KV_W = 128
MASKHI = -65536
NW = 32
EPT = E // NW
BE = 40
NBLK = EPT // BE
GSZ = 10
GPB = BE // GSZ
ROWS_PER_SUB = NP // 16
RB = 632
GRID = NP // RB
INV_SQRT_D = 1.0 / math.sqrt(float(D))

_f32 = jnp.float32
_i32 = jnp.int32


def _sc_edge_body(qe_hbm, kv_hbm, src_hbm, dst_hbm, attrp_hbm, out_hbm,
                  acc_sh, dst_b, src_b, qe_b, kv_b, attr_b, msg_b, scat_b,
                  isem0, isem1, gsem0, gsem1, ssem0, ssem1):
    c = lax.axis_index("c")
    s = lax.axis_index("s")
    wid = s * 2 + c
    isem = (isem0, isem1)
    gsem = (gsem0, gsem1)
    ssem = (ssem0, ssem1)

    def _zrow(i, _):
        msg_b[0, i // 9, pl.ds((i % 9) * 16, 16)] = jnp.zeros((16,), _f32)
        return 0
    lax.fori_loop(0, BE * (ACC_W // 16), _zrow, 0)

    def _zcpy(j, _):
        pltpu.async_copy(msg_b.at[0],
                         acc_sh.at[pl.ds(s * ROWS_PER_SUB + j * BE, BE)],
                         ssem0)
        return 0
    lax.fori_loop(0, 15, _zcpy, 0)
    pltpu.async_copy(msg_b.at[0, pl.ds(0, 32)],
                     acc_sh.at[pl.ds(s * ROWS_PER_SUB + 600, 32)],
                     ssem1)

    def _zwait(j, _):
        pltpu.make_async_copy(
            msg_b.at[0],
            acc_sh.at[pl.ds(s * ROWS_PER_SUB + j * BE, BE)], ssem0).wait()
        return 0
    lax.fori_loop(0, 15, _zwait, 0)
    pltpu.make_async_copy(msg_b.at[0, pl.ds(0, 32)],
                          acc_sh.at[pl.ds(s * ROWS_PER_SUB + 600, 32)],
                          ssem1).wait()
    plsc.subcore_barrier()

    e_base = wid * EPT

    def _issue_s1(b, p):
        e0 = e_base + b * BE
        pltpu.async_copy(dst_hbm.at[pl.ds(e0, BE)], dst_b.at[p], isem[p])
        pltpu.async_copy(src_hbm.at[pl.ds(e0, BE)], src_b.at[p], isem[p])

    def _wait_s1(b, p):
        e0 = e_base + b * BE
        pltpu.make_async_copy(dst_hbm.at[pl.ds(e0, BE)], dst_b.at[p],
                              isem[p]).wait()
        pltpu.make_async_copy(src_hbm.at[pl.ds(e0, BE)], src_b.at[p],
                              isem[p]).wait()

    def _issue_s2(b, p):
        e0 = e_base + b * BE
        pltpu.async_copy(qe_hbm.at[dst_b.at[p]], qe_b.at[p], gsem[p])
        pltpu.async_copy(kv_hbm.at[src_b.at[p]], kv_b.at[p], gsem[p])
        pltpu.async_copy(attrp_hbm.at[pl.ds(e0, BE)], attr_b.at[p], gsem[p])

    def _wait_s2(b, p):
        e0 = e_base + b * BE
        pltpu.make_async_copy(qe_hbm.at[dst_b.at[p]], qe_b.at[p],
                              gsem[p]).wait()
        pltpu.make_async_copy(kv_hbm.at[src_b.at[p]], kv_b.at[p],
                              gsem[p]).wait()
        pltpu.make_async_copy(attrp_hbm.at[pl.ds(e0, BE)], attr_b.at[p],
                              gsem[p]).wait()

    def _scat_desc(p):
        return pltpu.make_async_copy(msg_b.at[p], acc_sh.at[scat_b.at[p]],
                                     ssem[p])

    e00 = e_base
    pltpu.sync_copy(dst_hbm.at[pl.ds(e00, BE)], dst_b.at[0])
    pltpu.sync_copy(src_hbm.at[pl.ds(e00, BE)], src_b.at[0])
    _issue_s2(0, 0)
    _issue_s1(1, 1)

    def _pair(t, _):
        for p in range(2):
            q = 1 - p
            b = 2 * t + p

            @pl.when(b < NBLK - 1)
            def _():
                _wait_s1(b + 1, q)
                _issue_s2(b + 1, q)

            @pl.when(b >= 2)
            def _():
                _scat_desc(p).wait()

            for j0 in (0, 16, 24):
                scat_b[p, pl.ds(j0, 16)] = dst_b[p, pl.ds(j0, 16)]

            @pl.when(b < NBLK - 2)
            def _():
                _issue_s1(b + 2, p)

            _wait_s2(b, p)

            def _grp(g, _g):
                eb = g * GSZ
                atts, accs = [], []
                for u in range(GSZ):
                    e = eb + u
                    att = attr_b[p, e, pl.ds(0, 16)]
                    qtl = lax.bitcast_convert_type(
                        qe_b[p, e, pl.ds(64, 16)], _f32)
                    acc = qtl * att
                    for j in range(4):
                        qw = qe_b[p, e, pl.ds(16 * j, 16)]
                        kw = kv_b[p, e, pl.ds(16 * j, 16)]
                        qlo = lax.bitcast_convert_type(qw << 16, _f32)
                        qhi = lax.bitcast_convert_type(qw & MASKHI, _f32)
                        klo = lax.bitcast_convert_type(kw << 16, _f32)
                        khi = lax.bitcast_convert_type(kw & MASKHI, _f32)
                        acc = acc + qlo * klo + qhi * khi
                    atts.append(att)
                    accs.append(acc)
                tots = [jnp.sum(a) for a in accs]
                exbs = [jnp.exp(jnp.full((16,), t * INV_SQRT_D, _f32))
                        for t in tots]
                for u in range(GSZ):
                    e = eb + u
                    exb = exbs[u]
                    for j in range(4):
                        vw = kv_b[p, e, pl.ds(64 + 16 * j, 16)]
                        vlo = lax.bitcast_convert_type(vw << 16, _f32)
                        vhi = lax.bitcast_convert_type(vw & MASKHI, _f32)
                        msg_b[p, e, pl.ds(16 * j, 16)] = exb * vlo
                        msg_b[p, e, pl.ds(64 + 16 * j, 16)] = exb * vhi
                    msg_b[p, e, pl.ds(128, 16)] = exb * atts[u]
                return 0
            lax.fori_loop(0, GPB, _grp, 0)
            _scat_desc(p).start(add=True)
        return 0
    lax.fori_loop(0, NBLK // 2, _pair, 0)

    _scat_desc(0).wait()
    _scat_desc(1).wait()
    plsc.subcore_barrier()
    pltpu.sync_copy(acc_sh.at[pl.ds(s * ROWS_PER_SUB, ROWS_PER_SUB)],
                    out_hbm.at[c, pl.ds(s * ROWS_PER_SUB, ROWS_PER_SUB)])


def _make_sc_edge():
    mesh = plsc.VectorSubcoreMesh(core_axis_name="c", subcore_axis_name="s")
    return functools.partial(
        pl.kernel,
        out_type=jax.ShapeDtypeStruct((2, NP, ACC_W), _f32),
        mesh=mesh,
        compiler_params=pltpu.CompilerParams(needs_layout_passes=False,
                                             use_tc_tiling_on_sc=False),
        scratch_types=[
            pltpu.VMEM_SHARED((NP, ACC_W), _f32),
            pltpu.VMEM((2, BE), _i32),
            pltpu.VMEM((2, BE), _i32),
            pltpu.VMEM((2, BE, QE_W), _i32),
            pltpu.VMEM((2, BE, KV_W), _i32),
            pltpu.VMEM((2, BE, 16), _f32),
            pltpu.VMEM((2, BE, ACC_W), _f32),
            pltpu.VMEM((2, BE), _i32),
            pltpu.SemaphoreType.DMA,
            pltpu.SemaphoreType.DMA,
            pltpu.SemaphoreType.DMA,
            pltpu.SemaphoreType.DMA,
            pltpu.SemaphoreType.DMA,
            pltpu.SemaphoreType.DMA,
        ],
    )(_sc_edge_body)


_sc_edge = _make_sc_edge()


def _pack_bf16(x):
    u = lax.bitcast_convert_type(x.astype(jnp.bfloat16), jnp.uint16)
    ui = u.astype(_i32)
    return ui[:, 0:64] | (ui[:, 64:128] << 16)


def _emit_packed(hn, w3_ref, b3_ref, wt_ref, bt_ref, qe_ref, kv_ref):
    qkv = jnp.dot(hn, w3_ref[...], preferred_element_type=_f32) + b3_ref[...]
    tl = jnp.dot(hn, wt_ref[...], preferred_element_type=_f32) + bt_ref[...]
    qe_ref[...] = jnp.concatenate(
        [_pack_bf16(qkv[:, 0:128]), lax.bitcast_convert_type(tl, _i32)], axis=1)
    kv_ref[...] = jnp.concatenate(
        [_pack_bf16(qkv[:, 128:256]), _pack_bf16(qkv[:, 256:384])], axis=1)


def _proj0_body(h_ref, w3_ref, b3_ref, wt_ref, bt_ref, qe_ref, kv_ref):
    _emit_packed(h_ref[...], w3_ref, b3_ref, wt_ref, bt_ref, qe_ref, kv_ref)


def _proj0(h, w3, b3, wt, bt):
    return pl.pallas_call(
        _proj0_body,
        grid=(GRID,),
        in_specs=[
            pl.BlockSpec((RB, D), lambda i: (i, 0)),
            pl.BlockSpec((D, 384), lambda i: (0, 0)),
            pl.BlockSpec((1, 384), lambda i: (0, 0)),
            pl.BlockSpec((D, 16), lambda i: (0, 0)),
            pl.BlockSpec((1, 16), lambda i: (0, 0)),
        ],
        out_specs=[
            pl.BlockSpec((RB, QE_W), lambda i: (i, 0)),
            pl.BlockSpec((RB, KV_W), lambda i: (i, 0)),
        ],
        out_shape=[
            jax.ShapeDtypeStruct((NP, QE_W), _i32),
            jax.ShapeDtypeStruct((NP, KV_W), _i32),
        ],
    )(h, w3, b3, wt, bt)


def _upd_block(a0_ref, a1_ref, h_ref, ws_ref, bs_ref, p_ref):
    a = a0_ref[...] + a1_ref[...]
    t = jnp.dot(a[:, 128:144], p_ref[...], preferred_element_type=_f32)
    hs = jnp.dot(h_ref[...], ws_ref[...], preferred_element_type=_f32) + bs_ref[...]
    hn = (a[:, 0:128] + t[:, 0:128]) / (t[:, 128:256] + 1e-30) + hs
    return jnp.maximum(hn, 0.0)


def _layer_body(a0_ref, a1_ref, h_ref, ws_ref, bs_ref, p_ref,
                w3_ref, b3_ref, wt_ref, bt_ref, hn_ref, qe_ref, kv_ref):
    hn = _upd_block(a0_ref, a1_ref, h_ref, ws_ref, bs_ref, p_ref)
    hn_ref[...] = hn
    _emit_packed(hn, w3_ref, b3_ref, wt_ref, bt_ref, qe_ref, kv_ref)


def _layer(a0, a1, h, ws, bs, pm, w3, b3, wt, bt):
    return pl.pallas_call(
        _layer_body,
        grid=(GRID,),
        in_specs=[
            pl.BlockSpec((RB, ACC_W), lambda i: (i, 0)),
            pl.BlockSpec((RB, ACC_W), lambda i: (i, 0)),
            pl.BlockSpec((RB, D), lambda i: (i, 0)),
            pl.BlockSpec((D, D), lambda i: (0, 0)),
            pl.BlockSpec((1, D), lambda i: (0, 0)),
            pl.BlockSpec((16, 256), lambda i: (0, 0)),
            pl.BlockSpec((D, 384), lambda i: (0, 0)),
            pl.BlockSpec((1, 384), lambda i: (0, 0)),
            pl.BlockSpec((D, 16), lambda i: (0, 0)),
            pl.BlockSpec((1, 16), lambda i: (0, 0)),
        ],
        out_specs=[
            pl.BlockSpec((RB, D), lambda i: (i, 0)),
            pl.BlockSpec((RB, QE_W), lambda i: (i, 0)),
            pl.BlockSpec((RB, KV_W), lambda i: (i, 0)),
        ],
        out_shape=[
            jax.ShapeDtypeStruct((NP, D), _f32),
            jax.ShapeDtypeStruct((NP, QE_W), _i32),
            jax.ShapeDtypeStruct((NP, KV_W), _i32),
        ],
    )(a0, a1, h, ws, bs, pm, w3, b3, wt, bt)


def _head_body(a0_ref, a1_ref, h_ref, ws_ref, bs_ref, p_ref, oh_ref,
               w0_ref, b0_ref, w1_ref, b1_ref, w3_ref, b3_ref,
               out_ref, sums, cnts):
    i = pl.program_id(0)

    @pl.when(i == 0)
    def _():
        sums[...] = jnp.zeros((64, D), _f32)
        cnts[...] = jnp.zeros((64, D), _f32)

    oh = oh_ref[...]
    h = _upd_block(a0_ref, a1_ref, h_ref, ws_ref, bs_ref, p_ref)
    dn = (((0,), (0,)), ((), ()))
    sums[...] += lax.dot_general(oh, h, dn, preferred_element_type=_f32)
    cnts[...] += lax.dot_general(oh, jnp.ones_like(h), dn,
                                 preferred_element_type=_f32)

    @pl.when(i == GRID - 1)
    def _():
        g = sums[...] / jnp.maximum(cnts[...], 1.0)
        g = jnp.maximum(jnp.dot(g, w0_ref[...], preferred_element_type=_f32)
                        + b0_ref[...], 0.0)
        g = jnp.maximum(jnp.dot(g, w1_ref[...], preferred_element_type=_f32)
                        + b1_ref[...], 0.0)
        logits = jnp.dot(g, w3_ref[...], preferred_element_type=_f32) + b3_ref[...]
        mask2 = lax.broadcasted_iota(_i32, (64, D), 1) < 2
        neg = jnp.where(mask2, logits, -1e30)
        m = jnp.max(neg, axis=1, keepdims=True)
        lse = jnp.log(jnp.sum(jnp.where(mask2, jnp.exp(neg - m), 0.0),
                              axis=1, keepdims=True)) + m
        out_ref[...] = (logits - lse)[:, 0:2]


def _head(a0, a1, h, ws, bs, pm, oh, w0, b0, w1, b1, w3, b3):
    return pl.pallas_call(
        _head_body,
        grid=(GRID,),
        in_specs=[
            pl.BlockSpec((RB, ACC_W), lambda i: (i, 0)),
            pl.BlockSpec((RB, ACC_W), lambda i: (i, 0)),
            pl.BlockSpec((RB, D), lambda i: (i, 0)),
            pl.BlockSpec((D, D), lambda i: (0, 0)),
            pl.BlockSpec((1, D), lambda i: (0, 0)),
            pl.BlockSpec((16, 256), lambda i: (0, 0)),
            pl.BlockSpec((RB, 64), lambda i: (i, 0)),
            pl.BlockSpec((D, D), lambda i: (0, 0)),
            pl.BlockSpec((1, D), lambda i: (0, 0)),
            pl.BlockSpec((D, D), lambda i: (0, 0)),
            pl.BlockSpec((1, D), lambda i: (0, 0)),
            pl.BlockSpec((D, D), lambda i: (0, 0)),
            pl.BlockSpec((1, D), lambda i: (0, 0)),
        ],
        out_specs=pl.BlockSpec((64, 2), lambda i: (0, 0)),
        out_shape=jax.ShapeDtypeStruct((64, 2), _f32),
        scratch_shapes=[
            pltpu.VMEM((64, D), _f32),
            pltpu.VMEM((64, D), _f32),
        ],
    )(a0, a1, h, ws, bs, pm, oh, w0, b0, w1, b1, w3, b3)


def kernel(x, edge_index, edge_attr, flexible_idx, batchs, params):
    src = edge_index[0]
    dst = edge_index[1]
    attrp = jnp.concatenate(
        [jnp.ones((E, 1), _f32), edge_attr, jnp.zeros((E, 12), _f32)], axis=1)
    oh = (batchs[:, None] == jnp.arange(64, dtype=_i32)[None, :]).astype(_f32)
    oh = jnp.concatenate([oh, jnp.zeros((NP - N, 64), _f32)], axis=0)

    h = jnp.concatenate([x, jnp.zeros((NP - N, D), _f32)], axis=0)
    p = params

    def _wts(l):
        wq, bq = p['conv%d_Wq' % l], p['conv%d_bq' % l]
        wk, bk = p['conv%d_Wk' % l], p['conv%d_bk' % l]
        wv, bv = p['conv%d_Wv' % l], p['conv%d_bv' % l]
        ws, bs = p['conv%d_Ws' % l], p['conv%d_bs' % l]
        we = p['conv%d_We' % l]
        wet = we.T
        w3 = jnp.concatenate([wq, wk, wv], axis=1)
        b3 = jnp.concatenate([bq, bk, bv])[None, :]
        wt = jnp.concatenate(
            [jnp.zeros((D, 1), _f32), wq @ wet, jnp.zeros((D, 12), _f32)],
            axis=1)
        bt = jnp.concatenate(
            [jnp.zeros((1,), _f32), bq @ wet, jnp.zeros((12,), _f32)]
        )[None, :]
        pm = jnp.zeros((16, 256), _f32)
        pm = pm.at[1:4, 0:128].set(we)
        pm = pm.at[0, 128:256].set(1.0)
        return w3, b3, wt, bt, ws, bs[None, :], pm

    w3_0, b3_0, wt_0, bt_0, ws_0, bs_0, pm_0 = _wts(0)
    w3_1, b3_1, wt_1, bt_1, ws_1, bs_1, pm_1 = _wts(1)
    w3_2, b3_2, wt_2, bt_2, ws_2, bs_2, pm_2 = _wts(2)

    qe_pk, kv_pk = _proj0(h, w3_0, b3_0, wt_0, bt_0)
    acc = _sc_edge(qe_pk, kv_pk, src, dst, attrp)
    h, qe_pk, kv_pk = _layer(acc[0], acc[1], h, ws_0, bs_0, pm_0,
                             w3_1, b3_1, wt_1, bt_1)
    acc = _sc_edge(qe_pk, kv_pk, src, dst, attrp)
    h, qe_pk, kv_pk = _layer(acc[0], acc[1], h, ws_1, bs_1, pm_1,
                             w3_2, b3_2, wt_2, bt_2)
    acc = _sc_edge(qe_pk, kv_pk, src, dst, attrp)

    return _head(acc[0], acc[1], h, ws_2, bs_2, pm_2, oh,
                 params['lin0_W'], params['lin0_b'][None, :],
                 params['lin1_W'], params['lin1_b'][None, :],
                 jnp.zeros((D, D), _f32).at[:, 0:2].set(params['lin3_W']),
                 jnp.zeros((1, D), _f32).at[0, 0:2].set(params['lin3_b']))


# speedup vs baseline: 1.0077x; 1.0077x over previous
import functools
import math

import jax
import jax.numpy as jnp
from jax import lax
from jax.experimental import pallas as pl
from jax.experimental.pallas import tpu as pltpu
from jax.experimental.pallas import tpu_sc as plsc

N = 10000
E = 320000
D = 128
NP = 10112
ACC_W = 144
QE_W = 80
KV_W = 128
MASKHI = -65536
NW = 32
EPT = E // NW
BE = 40
NBLK = EPT // BE
GSZ = 8
GPB = BE // GSZ
ROWS_PER_SUB = NP // 16
RB = 632
GRID = NP // RB
INV_SQRT_D = 1.0 / math.sqrt(float(D))

_f32 = jnp.float32
_i32 = jnp.int32


def _sc_edge_body(qe_hbm, kv_hbm, src_hbm, dst_hbm, attrp_hbm, out_hbm,
                  acc_sh, dst_b, src_b, qe_b, kv_b, attr_b, msg_b, scat_b,
                  isem0, isem1, gsem0, gsem1, ssem0, ssem1):
    c = lax.axis_index("c")
    s = lax.axis_index("s")
    wid = s * 2 + c
    isem = (isem0, isem1)
    gsem = (gsem0, gsem1)
    ssem = (ssem0, ssem1)

    def _zrow(i, _):
        msg_b[0, i // 9, pl.ds((i % 9) * 16, 16)] = jnp.zeros((16,), _f32)
        return 0
    lax.fori_loop(0, BE * (ACC_W // 16), _zrow, 0)

    def _zcpy(j, _):
        pltpu.async_copy(msg_b.at[0],
                         acc_sh.at[pl.ds(s * ROWS_PER_SUB + j * BE, BE)],
                         ssem0)
        return 0
    lax.fori_loop(0, 15, _zcpy, 0)
    pltpu.async_copy(msg_b.at[0, pl.ds(0, 32)],
                     acc_sh.at[pl.ds(s * ROWS_PER_SUB + 600, 32)],
                     ssem1)

    def _zwait(j, _):
        pltpu.make_async_copy(
            msg_b.at[0],
            acc_sh.at[pl.ds(s * ROWS_PER_SUB + j * BE, BE)], ssem0).wait()
        return 0
    lax.fori_loop(0, 15, _zwait, 0)
    pltpu.make_async_copy(msg_b.at[0, pl.ds(0, 32)],
                          acc_sh.at[pl.ds(s * ROWS_PER_SUB + 600, 32)],
                          ssem1).wait()
    plsc.subcore_barrier()

    e_base = wid * EPT

    def _issue_s1(b, p):
        e0 = e_base + b * BE
        pltpu.async_copy(dst_hbm.at[pl.ds(e0, BE)], dst_b.at[p], isem[p])
        pltpu.async_copy(src_hbm.at[pl.ds(e0, BE)], src_b.at[p], isem[p])

    def _wait_s1(b, p):
        e0 = e_base + b * BE
        pltpu.make_async_copy(dst_hbm.at[pl.ds(e0, BE)], dst_b.at[p],
                              isem[p]).wait()
        pltpu.make_async_copy(src_hbm.at[pl.ds(e0, BE)], src_b.at[p],
                              isem[p]).wait()

    def _issue_s2(b, p):
        e0 = e_base + b * BE
        pltpu.async_copy(qe_hbm.at[dst_b.at[p]], qe_b.at[p], gsem[p])
        pltpu.async_copy(kv_hbm.at[src_b.at[p]], kv_b.at[p], gsem[p])
        pltpu.async_copy(attrp_hbm.at[pl.ds(e0, BE)], attr_b.at[p], gsem[p])

    def _wait_s2(b, p):
        e0 = e_base + b * BE
        pltpu.make_async_copy(qe_hbm.at[dst_b.at[p]], qe_b.at[p],
                              gsem[p]).wait()
        pltpu.make_async_copy(kv_hbm.at[src_b.at[p]], kv_b.at[p],
                              gsem[p]).wait()
        pltpu.make_async_copy(attrp_hbm.at[pl.ds(e0, BE)], attr_b.at[p],
                              gsem[p]).wait()

    def _scat_desc(p):
        return pltpu.make_async_copy(msg_b.at[p], acc_sh.at[scat_b.at[p]],
                                     ssem[p])

    e00 = e_base
    pltpu.sync_copy(dst_hbm.at[pl.ds(e00, BE)], dst_b.at[0])
    pltpu.sync_copy(src_hbm.at[pl.ds(e00, BE)], src_b.at[0])
    _issue_s2(0, 0)
    _issue_s1(1, 1)

    def _pair(t, _):
        for p in range(2):
            q = 1 - p
            b = 2 * t + p

            @pl.when(b < NBLK - 1)
            def _():
                _wait_s1(b + 1, q)
                _issue_s2(b + 1, q)

            @pl.when(b >= 2)
            def _():
                _scat_desc(p).wait()

            for j0 in (0, 16, 24):
                scat_b[p, pl.ds(j0, 16)] = dst_b[p, pl.ds(j0, 16)]

            @pl.when(b < NBLK - 2)
            def _():
                _issue_s1(b + 2, p)

            _wait_s2(b, p)

            def _grp(g, _g):
                eb = g * GSZ
                atts, accs = [], []
                for u in range(GSZ):
                    e = eb + u
                    att = attr_b[p, e, pl.ds(0, 16)]
                    qtl = lax.bitcast_convert_type(
                        qe_b[p, e, pl.ds(64, 16)], _f32)
                    acc = qtl * att
                    for j in range(4):
                        qw = qe_b[p, e, pl.ds(16 * j, 16)]
                        kw = kv_b[p, e, pl.ds(16 * j, 16)]
                        qlo = lax.bitcast_convert_type(qw << 16, _f32)
                        qhi = lax.bitcast_convert_type(qw & MASKHI, _f32)
                        klo = lax.bitcast_convert_type(kw << 16, _f32)
                        khi = lax.bitcast_convert_type(kw & MASKHI, _f32)
                        acc = acc + qlo * klo + qhi * khi
                    atts.append(att)
                    accs.append(acc)
                tots = [jnp.sum(a) for a in accs]
                exbs = [jnp.exp(jnp.full((16,), t * INV_SQRT_D, _f32))
                        for t in tots]
                for u in range(GSZ):
                    e = eb + u
                    exb = exbs[u]
                    for j in range(4):
                        vw = kv_b[p, e, pl.ds(64 + 16 * j, 16)]
                        vlo = lax.bitcast_convert_type(vw << 16, _f32)
                        vhi = lax.bitcast_convert_type(vw & MASKHI, _f32)
                        msg_b[p, e, pl.ds(16 * j, 16)] = exb * vlo
                        msg_b[p, e, pl.ds(64 + 16 * j, 16)] = exb * vhi
                    msg_b[p, e, pl.ds(128, 16)] = exb * atts[u]
                return 0
            lax.fori_loop(0, GPB, _grp, 0)
            _scat_desc(p).start(add=True)
        return 0
    lax.fori_loop(0, NBLK // 2, _pair, 0)

    _scat_desc(0).wait()
    _scat_desc(1).wait()
    plsc.subcore_barrier()
    pltpu.sync_copy(acc_sh.at[pl.ds(s * ROWS_PER_SUB, ROWS_PER_SUB)],
                    out_hbm.at[c, pl.ds(s * ROWS_PER_SUB, ROWS_PER_SUB)])


def _make_sc_edge():
    mesh = plsc.VectorSubcoreMesh(core_axis_name="c", subcore_axis_name="s")
    return functools.partial(
        pl.kernel,
        out_type=jax.ShapeDtypeStruct((2, NP, ACC_W), _f32),
        mesh=mesh,
        compiler_params=pltpu.CompilerParams(needs_layout_passes=False,
                                             use_tc_tiling_on_sc=False),
        scratch_types=[
            pltpu.VMEM_SHARED((NP, ACC_W), _f32),
            pltpu.VMEM((2, BE), _i32),
            pltpu.VMEM((2, BE), _i32),
            pltpu.VMEM((2, BE, QE_W), _i32),
            pltpu.VMEM((2, BE, KV_W), _i32),
            pltpu.VMEM((2, BE, 16), _f32),
            pltpu.VMEM((2, BE, ACC_W), _f32),
            pltpu.VMEM((2, BE), _i32),
            pltpu.SemaphoreType.DMA,
            pltpu.SemaphoreType.DMA,
            pltpu.SemaphoreType.DMA,
            pltpu.SemaphoreType.DMA,
            pltpu.SemaphoreType.DMA,
            pltpu.SemaphoreType.DMA,
        ],
    )(_sc_edge_body)


_sc_edge = _make_sc_edge()


def _pack_bf16(x):
    u = lax.bitcast_convert_type(x.astype(jnp.bfloat16), jnp.uint16)
    ui = u.astype(_i32)
    return ui[:, 0:64] | (ui[:, 64:128] << 16)


def _emit_packed(hn, w3_ref, b3_ref, wt_ref, bt_ref, qe_ref, kv_ref):
    qkv = jnp.dot(hn, w3_ref[...], preferred_element_type=_f32) + b3_ref[...]
    tl = jnp.dot(hn, wt_ref[...], preferred_element_type=_f32) + bt_ref[...]
    qe_ref[...] = jnp.concatenate(
        [_pack_bf16(qkv[:, 0:128]), lax.bitcast_convert_type(tl, _i32)], axis=1)
    kv_ref[...] = jnp.concatenate(
        [_pack_bf16(qkv[:, 128:256]), _pack_bf16(qkv[:, 256:384])], axis=1)


def _proj0_body(h_ref, w3_ref, b3_ref, wt_ref, bt_ref, qe_ref, kv_ref):
    _emit_packed(h_ref[...], w3_ref, b3_ref, wt_ref, bt_ref, qe_ref, kv_ref)


def _proj0(h, w3, b3, wt, bt):
    return pl.pallas_call(
        _proj0_body,
        grid=(GRID,),
        in_specs=[
            pl.BlockSpec((RB, D), lambda i: (i, 0)),
            pl.BlockSpec((D, 384), lambda i: (0, 0)),
            pl.BlockSpec((1, 384), lambda i: (0, 0)),
            pl.BlockSpec((D, 16), lambda i: (0, 0)),
            pl.BlockSpec((1, 16), lambda i: (0, 0)),
        ],
        out_specs=[
            pl.BlockSpec((RB, QE_W), lambda i: (i, 0)),
            pl.BlockSpec((RB, KV_W), lambda i: (i, 0)),
        ],
        out_shape=[
            jax.ShapeDtypeStruct((NP, QE_W), _i32),
            jax.ShapeDtypeStruct((NP, KV_W), _i32),
        ],
    )(h, w3, b3, wt, bt)


def _upd_block(a0_ref, a1_ref, h_ref, ws_ref, bs_ref, p_ref):
    a = a0_ref[...] + a1_ref[...]
    t = jnp.dot(a[:, 128:144], p_ref[...], preferred_element_type=_f32)
    hs = jnp.dot(h_ref[...], ws_ref[...], preferred_element_type=_f32) + bs_ref[...]
    hn = (a[:, 0:128] + t[:, 0:128]) / (t[:, 128:256] + 1e-30) + hs
    return jnp.maximum(hn, 0.0)


def _layer_body(a0_ref, a1_ref, h_ref, ws_ref, bs_ref, p_ref,
                w3_ref, b3_ref, wt_ref, bt_ref, hn_ref, qe_ref, kv_ref):
    hn = _upd_block(a0_ref, a1_ref, h_ref, ws_ref, bs_ref, p_ref)
    hn_ref[...] = hn
    _emit_packed(hn, w3_ref, b3_ref, wt_ref, bt_ref, qe_ref, kv_ref)


def _layer(a0, a1, h, ws, bs, pm, w3, b3, wt, bt):
    return pl.pallas_call(
        _layer_body,
        grid=(GRID,),
        in_specs=[
            pl.BlockSpec((RB, ACC_W), lambda i: (i, 0)),
            pl.BlockSpec((RB, ACC_W), lambda i: (i, 0)),
            pl.BlockSpec((RB, D), lambda i: (i, 0)),
            pl.BlockSpec((D, D), lambda i: (0, 0)),
            pl.BlockSpec((1, D), lambda i: (0, 0)),
            pl.BlockSpec((16, 256), lambda i: (0, 0)),
            pl.BlockSpec((D, 384), lambda i: (0, 0)),
            pl.BlockSpec((1, 384), lambda i: (0, 0)),
            pl.BlockSpec((D, 16), lambda i: (0, 0)),
            pl.BlockSpec((1, 16), lambda i: (0, 0)),
        ],
        out_specs=[
            pl.BlockSpec((RB, D), lambda i: (i, 0)),
            pl.BlockSpec((RB, QE_W), lambda i: (i, 0)),
            pl.BlockSpec((RB, KV_W), lambda i: (i, 0)),
        ],
        out_shape=[
            jax.ShapeDtypeStruct((NP, D), _f32),
            jax.ShapeDtypeStruct((NP, QE_W), _i32),
            jax.ShapeDtypeStruct((NP, KV_W), _i32),
        ],
    )(a0, a1, h, ws, bs, pm, w3, b3, wt, bt)


def _head_body(a0_ref, a1_ref, h_ref, ws_ref, bs_ref, p_ref, oh_ref,
               w0_ref, b0_ref, w1_ref, b1_ref, w3_ref, b3_ref,
               out_ref, sums, cnts):
    i = pl.program_id(0)

    @pl.when(i == 0)
    def _():
        sums[...] = jnp.zeros((64, D), _f32)
        cnts[...] = jnp.zeros((64, D), _f32)

    oh = oh_ref[...]
    h = _upd_block(a0_ref, a1_ref, h_ref, ws_ref, bs_ref, p_ref)
    dn = (((0,), (0,)), ((), ()))
    sums[...] += lax.dot_general(oh, h, dn, preferred_element_type=_f32)
    cnts[...] += lax.dot_general(oh, jnp.ones_like(h), dn,
                                 preferred_element_type=_f32)

    @pl.when(i == GRID - 1)
    def _():
        g = sums[...] / jnp.maximum(cnts[...], 1.0)
        g = jnp.maximum(jnp.dot(g, w0_ref[...], preferred_element_type=_f32)
                        + b0_ref[...], 0.0)
        g = jnp.maximum(jnp.dot(g, w1_ref[...], preferred_element_type=_f32)
                        + b1_ref[...], 0.0)
        logits = jnp.dot(g, w3_ref[...], preferred_element_type=_f32) + b3_ref[...]
        mask2 = lax.broadcasted_iota(_i32, (64, D), 1) < 2
        neg = jnp.where(mask2, logits, -1e30)
        m = jnp.max(neg, axis=1, keepdims=True)
        lse = jnp.log(jnp.sum(jnp.where(mask2, jnp.exp(neg - m), 0.0),
                              axis=1, keepdims=True)) + m
        out_ref[...] = (logits - lse)[:, 0:2]


def _head(a0, a1, h, ws, bs, pm, oh, w0, b0, w1, b1, w3, b3):
    return pl.pallas_call(
        _head_body,
        grid=(GRID,),
        in_specs=[
            pl.BlockSpec((RB, ACC_W), lambda i: (i, 0)),
            pl.BlockSpec((RB, ACC_W), lambda i: (i, 0)),
            pl.BlockSpec((RB, D), lambda i: (i, 0)),
            pl.BlockSpec((D, D), lambda i: (0, 0)),
            pl.BlockSpec((1, D), lambda i: (0, 0)),
            pl.BlockSpec((16, 256), lambda i: (0, 0)),
            pl.BlockSpec((RB, 64), lambda i: (i, 0)),
            pl.BlockSpec((D, D), lambda i: (0, 0)),
            pl.BlockSpec((1, D), lambda i: (0, 0)),
            pl.BlockSpec((D, D), lambda i: (0, 0)),
            pl.BlockSpec((1, D), lambda i: (0, 0)),
            pl.BlockSpec((D, D), lambda i: (0, 0)),
            pl.BlockSpec((1, D), lambda i: (0, 0)),
        ],
        out_specs=pl.BlockSpec((64, 2), lambda i: (0, 0)),
        out_shape=jax.ShapeDtypeStruct((64, 2), _f32),
        scratch_shapes=[
            pltpu.VMEM((64, D), _f32),
            pltpu.VMEM((64, D), _f32),
        ],
    )(a0, a1, h, ws, bs, pm, oh, w0, b0, w1, b1, w3, b3)


def kernel(x, edge_index, edge_attr, flexible_idx, batchs, params):
    src = edge_index[0]
    dst = edge_index[1]
    attrp = jnp.concatenate(
        [jnp.ones((E, 1), _f32), edge_attr, jnp.zeros((E, 12), _f32)], axis=1)
    oh = (batchs[:, None] == jnp.arange(64, dtype=_i32)[None, :]).astype(_f32)
    oh = jnp.concatenate([oh, jnp.zeros((NP - N, 64), _f32)], axis=0)

    h = jnp.concatenate([x, jnp.zeros((NP - N, D), _f32)], axis=0)
    p = params

    def _wts(l):
        wq, bq = p['conv%d_Wq' % l], p['conv%d_bq' % l]
        wk, bk = p['conv%d_Wk' % l], p['conv%d_bk' % l]
        wv, bv = p['conv%d_Wv' % l], p['conv%d_bv' % l]
        ws, bs = p['conv%d_Ws' % l], p['conv%d_bs' % l]
        we = p['conv%d_We' % l]
        wet = we.T
        w3 = jnp.concatenate([wq, wk, wv], axis=1)
        b3 = jnp.concatenate([bq, bk, bv])[None, :]
        wt = jnp.concatenate(
            [jnp.zeros((D, 1), _f32), wq @ wet, jnp.zeros((D, 12), _f32)],
            axis=1)
        bt = jnp.concatenate(
            [jnp.zeros((1,), _f32), bq @ wet, jnp.zeros((12,), _f32)]
        )[None, :]
        pm = jnp.zeros((16, 256), _f32)
        pm = pm.at[1:4, 0:128].set(we)
        pm = pm.at[0, 128:256].set(1.0)
        return w3, b3, wt, bt, ws, bs[None, :], pm

    w3_0, b3_0, wt_0, bt_0, ws_0, bs_0, pm_0 = _wts(0)
    w3_1, b3_1, wt_1, bt_1, ws_1, bs_1, pm_1 = _wts(1)
    w3_2, b3_2, wt_2, bt_2, ws_2, bs_2, pm_2 = _wts(2)

    qe_pk, kv_pk = _proj0(h, w3_0, b3_0, wt_0, bt_0)
    acc = _sc_edge(qe_pk, kv_pk, src, dst, attrp)
    h, qe_pk, kv_pk = _layer(acc[0], acc[1], h, ws_0, bs_0, pm_0,
                             w3_1, b3_1, wt_1, bt_1)
    acc = _sc_edge(qe_pk, kv_pk, src, dst, attrp)
    h, qe_pk, kv_pk = _layer(acc[0], acc[1], h, ws_1, bs_1, pm_1,
                             w3_2, b3_2, wt_2, bt_2)
    acc = _sc_edge(qe_pk, kv_pk, src, dst, attrp)

    return _head(acc[0], acc[1], h, ws_2, bs_2, pm_2, oh,
                 params['lin0_W'], params['lin0_b'][None, :],
                 params['lin1_W'], params['lin1_b'][None, :],
                 jnp.zeros((D, D), _f32).at[:, 0:2].set(params['lin3_W']),
                 jnp.zeros((1, D), _f32).at[0, 0:2].set(params['lin3_b']))
